# baseline (speedup 1.0000x reference)
import functools

import jax
import jax.numpy as jnp
from jax import lax
from jax.experimental import pallas as pl
from jax.experimental.pallas import tpu as pltpu
from jax.experimental.pallas import tpu_sc as plsc

EPS = 1e-6
NC = 2
NS = 16
NW = NC * NS
L = 16

CHUNK = 160
DMA_ROWS = 128
P_PITCH = L + 1
Q_OFF = L * P_PITCH


def _rsqrt(v):
    i = plsc.bitcast(v, jnp.int32)
    y = plsc.bitcast(jnp.int32(0x5F3759DF) - (i >> 1), jnp.float32)
    for _ in range(3):
        y = y * (1.5 - (0.5 * v) * y * y)
    return y


def _make_kernel(n_b, n_l, hidden):
    assert hidden == 4 * L
    n_rows = n_b * n_l
    rows_per_w = n_rows // NW
    assert rows_per_w * NW == n_rows
    n_pairs = rows_per_w // (2 * CHUNK)
    assert n_pairs * 2 * CHUNK == rows_per_w and CHUNK % L == 0
    import math
    piece = math.gcd(CHUNK, n_l)
    assert piece % 8 == 0

    mesh = plsc.VectorSubcoreMesh(core_axis_name="c", subcore_axis_name="s")

    @functools.partial(
        pl.kernel,
        out_type=jax.ShapeDtypeStruct((n_b, n_l, hidden), jnp.float32),
        mesh=mesh,
        scratch_types=[
            pltpu.VMEM((2 * CHUNK,), jnp.int32),
            pltpu.VMEM((CHUNK, 2 * hidden), jnp.float32),
            pltpu.VMEM((CHUNK, 2 * hidden), jnp.float32),
            pltpu.VMEM((CHUNK, hidden), jnp.float32),
            pltpu.VMEM((CHUNK, hidden), jnp.float32),
            pltpu.VMEM((hidden,), jnp.float32),
            pltpu.VMEM((hidden,), jnp.float32),
            pltpu.VMEM((2 * L * P_PITCH,), jnp.float32),
            pltpu.SemaphoreType.DMA,
            pltpu.SemaphoreType.DMA,
        ],
        compiler_params=pltpu.CompilerParams(needs_layout_passes=False),
    )
    def kern(x_ref, tab_ref, al_ref, be_ref, out_ref,
             idx_v, rows_a, rows_b, out_a, out_b, al_v, be_v, p_v,
             semg, semo):
        wid = lax.axis_index("s") * NC + lax.axis_index("c")
        pltpu.sync_copy(al_ref, al_v)
        pltpu.sync_copy(be_ref, be_v)
        a_vecs = [al_v[pl.ds(k * L, L)] for k in range(hidden // L)]
        b_vecs = [be_v[pl.ds(k * L, L)] for k in range(hidden // L)]
        iota = lax.iota(jnp.int32, L)
        iota_p = iota * P_PITCH

        def gather(half, buf):
            hs = []
            for off in range(0, CHUNK, DMA_ROWS):
                n = min(DMA_ROWS, CHUNK - off)
                hs.append(pltpu.async_copy(
                    tab_ref.at[idx_v.at[pl.ds(half * CHUNK + off, n)]],
                    buf.at[pl.ds(off, n)],
                    semg,
                ))
            return hs

        def process(buf, dst):
            def group_body(g, carry2):
                r0 = g * L
                for l in range(L):
                    v = [buf[r0 + l, pl.ds(k * L, L)]
                         for k in range(hidden // L)]
                    s_l = (v[0] + v[1]) + (v[2] + v[3])
                    q_l = (v[0] * v[0] + v[1] * v[1]) + (v[2] * v[2] + v[3] * v[3])
                    sidx = iota + (P_PITCH * l)
                    plsc.store_scatter(p_v, [sidx], s_l)
                    plsc.store_scatter(p_v, [sidx + Q_OFF], q_l)
                s_acc = [jnp.zeros((L,), jnp.float32) for _ in range(4)]
                q_acc = [jnp.zeros((L,), jnp.float32) for _ in range(4)]
                for c in range(L):
                    gv = plsc.load_gather(p_v, [iota_p + c])
                    hv = plsc.load_gather(p_v, [iota_p + (Q_OFF + c)])
                    s_acc[c % 4] = s_acc[c % 4] + gv
                    q_acc[c % 4] = q_acc[c % 4] + hv
                s = (s_acc[0] + s_acc[1]) + (s_acc[2] + s_acc[3])
                ss = (q_acc[0] + q_acc[1]) + (q_acc[2] + q_acc[3])
                mean = s * (1.0 / hidden)
                var = jnp.maximum((ss - s * mean) * (1.0 / (hidden - 1)),
                                  jnp.float32(0.0))
                std = var * _rsqrt(var)
                inv = 1.0 / (std + EPS)
                for l in range(L):
                    m_l = mean[l]
                    i_l = inv[l]
                    for k in range(hidden // L):
                        v = buf[r0 + l, pl.ds(k * L, L)]
                        o = a_vecs[k] * ((v - m_l) * i_l + b_vecs[k])
                        dst[r0 + l, pl.ds(k * L, L)] = o
                return carry2

            lax.fori_loop(0, CHUNK // L, group_body, 0)

        def write_out(src, base):
            hs = []
            for off in range(0, CHUNK, piece):
                fr = base + off
                bi = fr // n_l
                ti = fr - bi * n_l
                hs.append(pltpu.async_copy(
                    src.at[pl.ds(off, piece)],
                    out_ref.at[bi].at[pl.ds(ti, piece)],
                    semo,
                ))
            return hs

        def pair_body(pi, carry):
            base = wid * rows_per_w + pi * (2 * CHUNK)
            pltpu.sync_copy(x_ref.at[pl.ds(base, 2 * CHUNK)], idx_v)
            g_a = gather(0, rows_a)
            g_b = gather(1, rows_b)
            for h in g_a:
                h.wait()
            process(rows_a, out_a)
            o_a = write_out(out_a, base)
            for h in g_b:
                h.wait()
            process(rows_b, out_b)
            o_b = write_out(out_b, base + CHUNK)
            for h in o_a:
                h.wait()
            for h in o_b:
                h.wait()
            return carry

        lax.fori_loop(0, n_pairs, pair_body, 0)

    return kern


def _pad_block(t_ref, o_ref):
    t = t_ref[...]
    o_ref[...] = jnp.concatenate([t, jnp.zeros_like(t)], axis=1)


def _pad_table(table):
    vocab, hidden = table.shape
    br = 2000
    while vocab % br or br % 8:
        br -= 8
    return pl.pallas_call(
        _pad_block,
        grid=(vocab // br,),
        in_specs=[pl.BlockSpec((br, hidden), lambda i: (i, 0))],
        out_specs=pl.BlockSpec((br, 2 * hidden), lambda i: (i, 0)),
        out_shape=jax.ShapeDtypeStruct((vocab, 2 * hidden), jnp.float32),
    )(table)


def kernel(x, table, alpha, beta):
    b, l = x.shape
    vocab, hidden = table.shape
    x_flat = x.reshape(-1).astype(jnp.int32)
    table_p = _pad_table(table)
    kern = _make_kernel(b, l, hidden)
    return kern(x_flat, table_p, alpha, beta)


# speedup vs baseline: 1.1082x; 1.1082x over previous
import functools

import jax
import jax.numpy as jnp
from jax import lax
from jax.experimental import pallas as pl
from jax.experimental.pallas import tpu as pltpu
from jax.experimental.pallas import tpu_sc as plsc

EPS = 1e-6
NC = 2
NS = 16
NW = NC * NS
L = 16

CHUNK = 160
DMA_ROWS = 128
P_PITCH = L + 1
Q_OFF = L * P_PITCH


def _rsqrt(v):
    i = plsc.bitcast(v, jnp.int32)
    y = plsc.bitcast(jnp.int32(0x5F3759DF) - (i >> 1), jnp.float32)
    for _ in range(3):
        y = y * (1.5 - (0.5 * v) * y * y)
    return y


def _make_kernel(n_rows, hidden):
    assert hidden == 4 * L
    rows_per_w = n_rows // NW
    assert rows_per_w * NW == n_rows
    n_pairs = rows_per_w // (2 * CHUNK)
    assert n_pairs * 2 * CHUNK == rows_per_w and CHUNK % L == 0

    mesh = plsc.VectorSubcoreMesh(core_axis_name="c", subcore_axis_name="s")

    @functools.partial(
        pl.kernel,
        out_type=jax.ShapeDtypeStruct((n_rows, hidden), jnp.float32),
        mesh=mesh,
        scratch_types=[
            pltpu.VMEM((2 * CHUNK,), jnp.int32),
            pltpu.VMEM((CHUNK, 2 * hidden), jnp.float32),
            pltpu.VMEM((CHUNK, 2 * hidden), jnp.float32),
            pltpu.VMEM((CHUNK, hidden), jnp.float32),
            pltpu.VMEM((CHUNK, hidden), jnp.float32),
            pltpu.VMEM((hidden,), jnp.float32),
            pltpu.VMEM((hidden,), jnp.float32),
            pltpu.VMEM((2 * L * P_PITCH,), jnp.float32),
            pltpu.SemaphoreType.DMA,
            pltpu.SemaphoreType.DMA,
        ],
        compiler_params=pltpu.CompilerParams(needs_layout_passes=False),
    )
    def kern(x_ref, tab_ref, al_ref, be_ref, out_ref,
             idx_v, rows_a, rows_b, out_a, out_b, al_v, be_v, p_v,
             semg, semo):
        wid = lax.axis_index("s") * NC + lax.axis_index("c")
        pltpu.sync_copy(al_ref, al_v)
        pltpu.sync_copy(be_ref, be_v)
        a_vecs = [al_v[pl.ds(k * L, L)] for k in range(hidden // L)]
        b_vecs = [be_v[pl.ds(k * L, L)] for k in range(hidden // L)]
        iota = lax.iota(jnp.int32, L)
        iota_p = iota * P_PITCH

        def gather(half, buf):
            hs = []
            for off in range(0, CHUNK, DMA_ROWS):
                n = min(DMA_ROWS, CHUNK - off)
                hs.append(pltpu.async_copy(
                    tab_ref.at[idx_v.at[pl.ds(half * CHUNK + off, n)]],
                    buf.at[pl.ds(off, n)],
                    semg,
                ))
            return hs

        def process(buf, dst):
            def group_body(g, carry2):
                r0 = g * L
                for l in range(L):
                    v = [buf[r0 + l, pl.ds(k * L, L)]
                         for k in range(hidden // L)]
                    s_l = (v[0] + v[1]) + (v[2] + v[3])
                    q_l = (v[0] * v[0] + v[1] * v[1]) + (v[2] * v[2] + v[3] * v[3])
                    sidx = iota + (P_PITCH * l)
                    plsc.store_scatter(p_v, [sidx], s_l)
                    plsc.store_scatter(p_v, [sidx + Q_OFF], q_l)
                s_acc = [jnp.zeros((L,), jnp.float32) for _ in range(4)]
                q_acc = [jnp.zeros((L,), jnp.float32) for _ in range(4)]
                for c in range(L):
                    gv = plsc.load_gather(p_v, [iota_p + c])
                    hv = plsc.load_gather(p_v, [iota_p + (Q_OFF + c)])
                    s_acc[c % 4] = s_acc[c % 4] + gv
                    q_acc[c % 4] = q_acc[c % 4] + hv
                s = (s_acc[0] + s_acc[1]) + (s_acc[2] + s_acc[3])
                ss = (q_acc[0] + q_acc[1]) + (q_acc[2] + q_acc[3])
                mean = s * (1.0 / hidden)
                var = jnp.maximum((ss - s * mean) * (1.0 / (hidden - 1)),
                                  jnp.float32(0.0))
                std = var * _rsqrt(var)
                inv = 1.0 / (std + EPS)
                for l in range(L):
                    m_l = mean[l]
                    i_l = inv[l]
                    for k in range(hidden // L):
                        v = buf[r0 + l, pl.ds(k * L, L)]
                        o = a_vecs[k] * ((v - m_l) * i_l + b_vecs[k])
                        dst[r0 + l, pl.ds(k * L, L)] = o
                return carry2

            lax.fori_loop(0, CHUNK // L, group_body, 0)

        def pair_body(pi, carry):
            base = wid * rows_per_w + pi * (2 * CHUNK)
            pltpu.sync_copy(x_ref.at[pl.ds(base, 2 * CHUNK)], idx_v)
            g_a = gather(0, rows_a)
            g_b = gather(1, rows_b)
            for h in g_a:
                h.wait()
            process(rows_a, out_a)
            o_a = pltpu.async_copy(out_a, out_ref.at[pl.ds(base, CHUNK)],
                                   semo)
            for h in g_b:
                h.wait()
            process(rows_b, out_b)
            o_b = pltpu.async_copy(out_b,
                                   out_ref.at[pl.ds(base + CHUNK, CHUNK)],
                                   semo)
            o_a.wait()
            o_b.wait()
            return carry

        lax.fori_loop(0, n_pairs, pair_body, 0)

    return kern


def _pad_block(t_ref, o_ref):
    t = t_ref[...]
    o_ref[...] = jnp.concatenate([t, jnp.zeros_like(t)], axis=1)


def _pad_table(table):
    vocab, hidden = table.shape
    br = 2000
    while vocab % br or br % 8:
        br -= 8
    return pl.pallas_call(
        _pad_block,
        grid=(vocab // br,),
        in_specs=[pl.BlockSpec((br, hidden), lambda i: (i, 0))],
        out_specs=pl.BlockSpec((br, 2 * hidden), lambda i: (i, 0)),
        out_shape=jax.ShapeDtypeStruct((vocab, 2 * hidden), jnp.float32),
    )(table)


def kernel(x, table, alpha, beta):
    b, l = x.shape
    vocab, hidden = table.shape
    x_flat = x.reshape(-1).astype(jnp.int32)
    table_p = _pad_table(table)
    kern = _make_kernel(b * l, hidden)
    out = kern(x_flat, table_p, alpha, beta)
    return out.reshape(b, l, hidden)
